# baseline (speedup 1.0000x reference)
import numpy as np
import jax
import jax.numpy as jnp
from jax.experimental import pallas as pl


def _enc_table(S, E):
    pos = np.arange(S, dtype=np.float64)[:, None]
    i = np.arange(E, dtype=np.float64)[None, :]
    angle = pos / np.power(10000.0, (i - np.mod(i, 2)) / E)
    enc = np.array(angle)
    enc[:, 0::2] = np.sin(angle[:, 0::2])
    enc[:, 1::2] = np.cos(angle[:, 1::2])
    return jnp.asarray(enc, dtype=jnp.float32)


def _body(x_ref, enc_ref, o_ref):
    xv = x_ref[...]
    o_ref[...] = jnp.where(xv == 0.0, xv, enc_ref[...][None])


def kernel(x):
    B, S, E = x.shape
    enc = _enc_table(S, E)
    BS = 256
    out = pl.pallas_call(
        _body,
        grid=(S // BS, B),
        in_specs=[
            pl.BlockSpec((1, BS, E), lambda s, b: (b, s, 0)),
            pl.BlockSpec((BS, E), lambda s, b: (s, 0)),
        ],
        out_specs=pl.BlockSpec((1, BS, E), lambda s, b: (b, s, 0)),
        out_shape=jax.ShapeDtypeStruct((B, S, E), jnp.float32),
    )(x, enc)
    return out
